# baseline (speedup 1.0000x reference)
import jax
import jax.numpy as jnp
from jax.experimental import pallas as pl
from jax.experimental.pallas import tpu as pltpu

_NB = 1024


def _chamfer_tc_kernel(x_ref, yt_ref, d1_ref, d2_ref):
    x = x_ref[0]
    yt = yt_ref[0]
    p = jax.lax.dot(x, -2.0 * yt, preferred_element_type=jnp.float32)
    x2 = jnp.sum(x * x, axis=1, keepdims=True)
    y2 = jnp.sum(yt * yt, axis=0, keepdims=True)
    d = (x2 + y2) + p

    d1_ref[0, 0, :] = jnp.maximum(jnp.min(d, axis=1), 0.0)

    colmin = jnp.min(d, axis=0)
    i = pl.program_id(1)

    @pl.when(i == 0)
    def _init():
        d2_ref[0, 0, :] = colmin

    @pl.when(i > 0)
    def _acc():
        d2_ref[0, 0, :] = jnp.minimum(d2_ref[0, 0, :], colmin)


def kernel(input1, input2):
    B, N, D = input1.shape
    M = input2.shape[1]
    yt = jnp.transpose(input2, (0, 2, 1))

    grid = (B, N // _NB)
    d1, d2 = pl.pallas_call(
        _chamfer_tc_kernel,
        grid=grid,
        in_specs=[
            pl.BlockSpec((1, _NB, D), lambda b, i: (b, i, 0)),
            pl.BlockSpec((1, D, M), lambda b, i: (b, 0, 0)),
        ],
        out_specs=[
            pl.BlockSpec((1, 1, _NB), lambda b, i: (b * (N // _NB) + i, 0, 0)),
            pl.BlockSpec((1, 1, M), lambda b, i: (b, 0, 0)),
        ],
        out_shape=[
            jax.ShapeDtypeStruct((B * (N // _NB), 1, _NB), jnp.float32),
            jax.ShapeDtypeStruct((B, 1, M), jnp.float32),
        ],
        compiler_params=pltpu.CompilerParams(
            dimension_semantics=("parallel", "arbitrary")),
    )(input1, yt)

    return (d1.reshape(B, N), jnp.maximum(d2.reshape(B, M), 0.0))
